# Initial kernel scaffold; baseline (speedup 1.0000x reference)
import jax
import jax.numpy as jnp
from jax.experimental import pallas as pl


def kernel(x, emb_weight):
    raise NotImplementedError("write your pallas kernel here")


# speedup vs baseline: 3.2496x; 3.2496x over previous
import jax
import jax.numpy as jnp
from jax.experimental import pallas as pl


_SEQ_BLK = 1024


def _add_kernel(x_ref, emb_ref, out_ref):
    out_ref[...] = x_ref[...] + emb_ref[...][None, :, :]


def kernel(x, emb_weight):
    batch, seq_len, d_model = x.shape
    seq_blk = min(_SEQ_BLK, seq_len)
    num_s = seq_len // seq_blk
    grid = (num_s, batch)
    return pl.pallas_call(
        _add_kernel,
        grid=grid,
        in_specs=[
            pl.BlockSpec((1, seq_blk, d_model), lambda s, b: (b, s, 0)),
            pl.BlockSpec((seq_blk, d_model), lambda s, b: (s, 0)),
        ],
        out_specs=pl.BlockSpec((1, seq_blk, d_model), lambda s, b: (b, s, 0)),
        out_shape=jax.ShapeDtypeStruct(x.shape, x.dtype),
    )(x, emb_weight)
